# baseline (speedup 1.0000x reference)
import jax
import jax.numpy as jnp
from jax.experimental import pallas as pl
from jax.experimental.pallas import tpu as pltpu

_T = 512


def _linoss_body(x_ref, btr_ref, bti_ref, ctr_ref, cti_ref, d_ref, ad_ref,
                 st_ref, o_ref, carry_ref):
    c = pl.program_id(0)
    i = pl.program_id(1)
    T = x_ref.shape[0]
    Ph = ad_ref.shape[1]

    a = jnp.maximum(ad_ref[...], 0.0)
    s = jax.nn.sigmoid(st_ref[...])
    mA = jnp.ones_like(s)
    mB = -s * a
    mC = s
    mD = 1.0 - s * s * a

    x = x_ref[...]
    f_r = jnp.dot(x, btr_ref[...] * s, preferred_element_type=jnp.float32)
    f_i = jnp.dot(x, bti_ref[...] * s, preferred_element_type=jnp.float32)

    @pl.when(i == 0)
    def _():
        carry_ref[...] = jnp.zeros_like(carry_ref)

    cc = carry_ref[...]
    c1r, c1i, c2r, c2i = cc[0:1], cc[1:2], cc[2:3], cc[3:4]
    d1r = mA * c1r + mB * c2r
    d1i = mA * c1i + mB * c2i
    d2r = mC * c1r + mD * c2r
    d2i = mC * c1i + mD * c2i

    rowmask = (jax.lax.broadcasted_iota(jnp.int32, (T, 1), 0) == 0
               ).astype(jnp.float32)
    b1r = f_r + rowmask * d1r
    b1i = f_i + rowmask * d1i
    b2r = f_r + rowmask * d2r
    b2i = f_i + rowmask * d2i

    nA, nB, nC, nD = mA, mB, mC, mD
    d = 1
    while d < T:
        z = jnp.zeros((d, Ph), jnp.float32)
        s1r = jnp.concatenate([z, b1r[:T - d]], axis=0)
        s1i = jnp.concatenate([z, b1i[:T - d]], axis=0)
        s2r = jnp.concatenate([z, b2r[:T - d]], axis=0)
        s2i = jnp.concatenate([z, b2i[:T - d]], axis=0)
        if d * 2 < T:
            nb1r = b1r + nA * s1r + nB * s2r
            nb1i = b1i + nA * s1i + nB * s2i
        else:
            r1r = (b1r[T - 1:T] + nA * b1r[T - 1 - d:T - d]
                   + nB * b2r[T - 1 - d:T - d])
            r1i = (b1i[T - 1:T] + nA * b1i[T - 1 - d:T - d]
                   + nB * b2i[T - 1 - d:T - d])
        b2r = b2r + nC * s1r + nD * s2r
        b2i = b2i + nC * s1i + nD * s2i
        if d * 2 < T:
            b1r, b1i = nb1r, nb1i
            tr = nA + nD
            nA2 = nA * nA + nB * nC
            nD2 = nD * nD + nB * nC
            nB, nC = nB * tr, nC * tr
            nA, nD = nA2, nD2
        d *= 2

    carry_ref[0:1] = r1r
    carry_ref[1:2] = r1i
    carry_ref[2:3] = b2r[T - 1:T]
    carry_ref[3:4] = b2i[T - 1:T]

    o = (jnp.dot(b2r, ctr_ref[...], preferred_element_type=jnp.float32)
         - jnp.dot(b2i, cti_ref[...], preferred_element_type=jnp.float32))
    o = o + jnp.where(c == 0, 1.0, 0.0) * (x * d_ref[...])
    o_ref[0] = o


def _sum_body(p_ref, o_ref):
    pp = p_ref[...]
    o_ref[...] = pp[0] + pp[1]


def kernel(input_sequence, A_diag_raw, B_real, B_img, C_real, C_img, D,
           steps_raw):
    L, H = input_sequence.shape
    P = A_diag_raw.shape[0]
    Ph = P // 2
    n_chunks = L // _T

    partials = pl.pallas_call(
        _linoss_body,
        out_shape=jax.ShapeDtypeStruct((2, L, H), jnp.float32),
        grid=(2, n_chunks),
        in_specs=[
            pl.BlockSpec((_T, H), lambda c, i: (i, 0)),
            pl.BlockSpec((H, Ph), lambda c, i: (0, c)),
            pl.BlockSpec((H, Ph), lambda c, i: (0, c)),
            pl.BlockSpec((Ph, H), lambda c, i: (c, 0)),
            pl.BlockSpec((Ph, H), lambda c, i: (c, 0)),
            pl.BlockSpec((1, H), lambda c, i: (0, 0)),
            pl.BlockSpec((1, Ph), lambda c, i: (0, c)),
            pl.BlockSpec((1, Ph), lambda c, i: (0, c)),
        ],
        out_specs=pl.BlockSpec((1, _T, H), lambda c, i: (c, i, 0)),
        scratch_shapes=[pltpu.VMEM((8, Ph), jnp.float32)],
        compiler_params=pltpu.CompilerParams(
            dimension_semantics=("parallel", "arbitrary"),
        ),
        name="linoss_scan",
    )(
        input_sequence,
        B_real.T, B_img.T,
        C_real.T, C_img.T,
        D.reshape(1, H),
        A_diag_raw.reshape(1, P),
        steps_raw.reshape(1, P),
    )

    T2 = L // 4
    out = pl.pallas_call(
        _sum_body,
        out_shape=jax.ShapeDtypeStruct((L, H), jnp.float32),
        grid=(2, 2),
        in_specs=[pl.BlockSpec((2, T2, H), lambda c, i: (0, 2 * c + i, 0))],
        out_specs=pl.BlockSpec((T2, H), lambda c, i: (2 * c + i, 0)),
        compiler_params=pltpu.CompilerParams(
            dimension_semantics=("parallel", "arbitrary"),
        ),
        name="linoss_sum",
    )(partials)
    return out


# speedup vs baseline: 1.1600x; 1.1600x over previous
import jax
import jax.numpy as jnp
from jax.experimental import pallas as pl
from jax.experimental.pallas import tpu as pltpu

_T = 512


def _linoss_body(x_ref, btr_ref, bti_ref, ctr_ref, cti_ref, d_ref, ad_ref,
                 st_ref, o_ref, carry_ref):
    i = pl.program_id(0)
    T = x_ref.shape[0]
    P = ad_ref.shape[1]

    a = jnp.maximum(ad_ref[...], 0.0)
    s = jax.nn.sigmoid(st_ref[...])
    mA = jnp.ones_like(s)
    mB = -s * a
    mC = s
    mD = 1.0 - s * s * a

    x = x_ref[...]
    f_r = jnp.dot(x, btr_ref[...] * s, preferred_element_type=jnp.float32)
    f_i = jnp.dot(x, bti_ref[...] * s, preferred_element_type=jnp.float32)

    @pl.when(i == 0)
    def _():
        carry_ref[...] = jnp.zeros_like(carry_ref)

    cc = carry_ref[...]
    c1r, c1i, c2r, c2i = cc[0:1], cc[1:2], cc[2:3], cc[3:4]
    d1r = mA * c1r + mB * c2r
    d1i = mA * c1i + mB * c2i
    d2r = mC * c1r + mD * c2r
    d2i = mC * c1i + mD * c2i

    rowmask = (jax.lax.broadcasted_iota(jnp.int32, (T, 1), 0) == 0
               ).astype(jnp.float32)
    b1r = f_r + rowmask * d1r
    b1i = f_i + rowmask * d1i
    b2r = f_r + rowmask * d2r
    b2i = f_i + rowmask * d2i

    nA, nB, nC, nD = mA, mB, mC, mD
    d = 1
    while d < T:
        z = jnp.zeros((d, P), jnp.float32)
        s1r = jnp.concatenate([z, b1r[:T - d]], axis=0)
        s1i = jnp.concatenate([z, b1i[:T - d]], axis=0)
        s2r = jnp.concatenate([z, b2r[:T - d]], axis=0)
        s2i = jnp.concatenate([z, b2i[:T - d]], axis=0)
        if d * 2 < T:
            nb1r = b1r + nA * s1r + nB * s2r
            nb1i = b1i + nA * s1i + nB * s2i
        else:
            r1r = (b1r[T - 1:T] + nA * b1r[T - 1 - d:T - d]
                   + nB * b2r[T - 1 - d:T - d])
            r1i = (b1i[T - 1:T] + nA * b1i[T - 1 - d:T - d]
                   + nB * b2i[T - 1 - d:T - d])
        b2r = b2r + nC * s1r + nD * s2r
        b2i = b2i + nC * s1i + nD * s2i
        if d * 2 < T:
            b1r, b1i = nb1r, nb1i
            tr = nA + nD
            nA2 = nA * nA + nB * nC
            nD2 = nD * nD + nB * nC
            nB, nC = nB * tr, nC * tr
            nA, nD = nA2, nD2
        d *= 2

    carry_ref[0:1] = r1r
    carry_ref[1:2] = r1i
    carry_ref[2:3] = b2r[T - 1:T]
    carry_ref[3:4] = b2i[T - 1:T]

    o = (jnp.dot(b2r, ctr_ref[...], preferred_element_type=jnp.float32)
         - jnp.dot(b2i, cti_ref[...], preferred_element_type=jnp.float32)
         + x * d_ref[...])
    o_ref[...] = o


def kernel(input_sequence, A_diag_raw, B_real, B_img, C_real, C_img, D,
           steps_raw):
    L, H = input_sequence.shape
    P = A_diag_raw.shape[0]
    n_chunks = L // _T

    return pl.pallas_call(
        _linoss_body,
        out_shape=jax.ShapeDtypeStruct((L, H), jnp.float32),
        grid=(n_chunks,),
        in_specs=[
            pl.BlockSpec((_T, H), lambda i: (i, 0)),
            pl.BlockSpec((H, P), lambda i: (0, 0)),
            pl.BlockSpec((H, P), lambda i: (0, 0)),
            pl.BlockSpec((P, H), lambda i: (0, 0)),
            pl.BlockSpec((P, H), lambda i: (0, 0)),
            pl.BlockSpec((1, H), lambda i: (0, 0)),
            pl.BlockSpec((1, P), lambda i: (0, 0)),
            pl.BlockSpec((1, P), lambda i: (0, 0)),
        ],
        out_specs=pl.BlockSpec((_T, H), lambda i: (i, 0)),
        scratch_shapes=[pltpu.VMEM((8, P), jnp.float32)],
        compiler_params=pltpu.CompilerParams(
            dimension_semantics=("arbitrary",),
        ),
        name="linoss_scan",
    )(
        input_sequence,
        B_real.T, B_img.T,
        C_real.T, C_img.T,
        D.reshape(1, H),
        A_diag_raw.reshape(1, P),
        steps_raw.reshape(1, P),
    )


# speedup vs baseline: 1.1899x; 1.0258x over previous
import jax
import jax.numpy as jnp
from jax.experimental import pallas as pl
from jax.experimental.pallas import tpu as pltpu

_T = 512


def _linoss_body(x_ref, btr_ref, bti_ref, ctr_ref, cti_ref, d_ref, ad_ref,
                 st_ref, o_ref, carry_ref, w2r_ref, w2i_ref, w3r_ref,
                 w3i_ref):
    i = pl.program_id(0)
    T = x_ref.shape[0]
    H = x_ref.shape[1]
    P = ad_ref.shape[1]
    Tp = T // 2

    a = jnp.maximum(ad_ref[...], 0.0)
    s = jax.nn.sigmoid(st_ref[...])
    mA = jnp.ones_like(s)
    mB = -s * a
    mC = s
    mD = 1.0 - s * s * a

    @pl.when(i == 0)
    def _():
        carry_ref[...] = jnp.zeros_like(carry_ref)
        bts_r = btr_ref[...] * s
        bts_i = bti_ref[...] * s
        zhp = jnp.zeros((H, P), jnp.float32)
        w2r_ref[...] = jnp.concatenate(
            [jnp.concatenate([bts_r, zhp], axis=1),
             jnp.concatenate([zhp, bts_r], axis=1)], axis=0)
        w2i_ref[...] = jnp.concatenate(
            [jnp.concatenate([bts_i, zhp], axis=1),
             jnp.concatenate([zhp, bts_i], axis=1)], axis=0)
        zph = jnp.zeros((P, H), jnp.float32)
        w3r_ref[...] = jnp.concatenate(
            [jnp.concatenate([ctr_ref[...], zph], axis=1),
             jnp.concatenate([zph, ctr_ref[...]], axis=1)], axis=0)
        w3i_ref[...] = jnp.concatenate(
            [jnp.concatenate([cti_ref[...], zph], axis=1),
             jnp.concatenate([zph, cti_ref[...]], axis=1)], axis=0)

    xp = x_ref[...].reshape(Tp, 2 * H)
    f_r = jnp.dot(xp, w2r_ref[...], preferred_element_type=jnp.float32)
    f_i = jnp.dot(xp, w2i_ref[...], preferred_element_type=jnp.float32)

    cc = carry_ref[...]
    c1r, c1i, c2r, c2i = cc[0:1], cc[1:2], cc[2:3], cc[3:4]
    zp = jnp.zeros((1, P), jnp.float32)
    d1r = jnp.concatenate([mA * c1r + mB * c2r, zp], axis=1)
    d1i = jnp.concatenate([mA * c1i + mB * c2i, zp], axis=1)
    d2r = jnp.concatenate([mC * c1r + mD * c2r, zp], axis=1)
    d2i = jnp.concatenate([mC * c1i + mD * c2i, zp], axis=1)
    rowmask = (jax.lax.broadcasted_iota(jnp.int32, (Tp, 1), 0) == 0
               ).astype(jnp.float32)
    g1r = f_r + rowmask * d1r
    g1i = f_i + rowmask * d1i
    g2r = f_r + rowmask * d2r
    g2i = f_i + rowmask * d2i

    lmask = jax.lax.broadcasted_iota(jnp.int32, (1, 2 * P), 1) < P
    zrow = jnp.zeros((1, 2 * P), jnp.float32)

    def fold_operand(g):
        sw = jnp.concatenate([g[:, P:], g[:, :P]], axis=1)
        swsh = jnp.concatenate([zrow, sw[:Tp - 1]], axis=0)
        return jnp.where(lmask, swsh, sw)

    h1r = fold_operand(g1r)
    h1i = fold_operand(g1i)
    h2r = fold_operand(g2r)
    h2i = fold_operand(g2i)

    tA = jnp.concatenate([mA, mA], axis=1)
    tB = jnp.concatenate([mB, mB], axis=1)
    tC = jnp.concatenate([mC, mC], axis=1)
    tD = jnp.concatenate([mD, mD], axis=1)

    b1r = g1r + tA * h1r + tB * h2r
    b1i = g1i + tA * h1i + tB * h2i
    b2r = g2r + tC * h1r + tD * h2r
    b2i = g2i + tC * h1i + tD * h2i

    qA = mA * mA + mB * mC
    qD = mD * mD + mB * mC
    qtr = mA + mD
    qB = mB * qtr
    qC = mC * qtr
    nA = jnp.concatenate([qA, qA], axis=1)
    nB = jnp.concatenate([qB, qB], axis=1)
    nC = jnp.concatenate([qC, qC], axis=1)
    nD = jnp.concatenate([qD, qD], axis=1)

    d = 1
    while d < Tp:
        z = jnp.zeros((d, 2 * P), jnp.float32)
        s1r = jnp.concatenate([z, b1r[:Tp - d]], axis=0)
        s1i = jnp.concatenate([z, b1i[:Tp - d]], axis=0)
        s2r = jnp.concatenate([z, b2r[:Tp - d]], axis=0)
        s2i = jnp.concatenate([z, b2i[:Tp - d]], axis=0)
        if d * 2 < Tp:
            nb1r = b1r + nA * s1r + nB * s2r
            nb1i = b1i + nA * s1i + nB * s2i
        else:
            r1r = (b1r[Tp - 1:Tp] + nA * b1r[Tp - 1 - d:Tp - d]
                   + nB * b2r[Tp - 1 - d:Tp - d])
            r1i = (b1i[Tp - 1:Tp] + nA * b1i[Tp - 1 - d:Tp - d]
                   + nB * b2i[Tp - 1 - d:Tp - d])
        b2r = b2r + nC * s1r + nD * s2r
        b2i = b2i + nC * s1i + nD * s2i
        if d * 2 < Tp:
            b1r, b1i = nb1r, nb1i
            tr2 = nA + nD
            nA2 = nA * nA + nB * nC
            nD2 = nD * nD + nB * nC
            nB, nC = nB * tr2, nC * tr2
            nA, nD = nA2, nD2
        d *= 2

    carry_ref[0:1] = r1r[:, P:]
    carry_ref[1:2] = r1i[:, P:]
    carry_ref[2:3] = b2r[Tp - 1:Tp, P:]
    carry_ref[3:4] = b2i[Tp - 1:Tp, P:]

    dd = jnp.concatenate([d_ref[...], d_ref[...]], axis=1)
    op = (jnp.dot(b2r, w3r_ref[...], preferred_element_type=jnp.float32)
          - jnp.dot(b2i, w3i_ref[...], preferred_element_type=jnp.float32)
          + xp * dd)
    o_ref[...] = op.reshape(T, H)


def kernel(input_sequence, A_diag_raw, B_real, B_img, C_real, C_img, D,
           steps_raw):
    L, H = input_sequence.shape
    P = A_diag_raw.shape[0]
    n_chunks = L // _T

    return pl.pallas_call(
        _linoss_body,
        out_shape=jax.ShapeDtypeStruct((L, H), jnp.float32),
        grid=(n_chunks,),
        in_specs=[
            pl.BlockSpec((_T, H), lambda i: (i, 0)),
            pl.BlockSpec((H, P), lambda i: (0, 0)),
            pl.BlockSpec((H, P), lambda i: (0, 0)),
            pl.BlockSpec((P, H), lambda i: (0, 0)),
            pl.BlockSpec((P, H), lambda i: (0, 0)),
            pl.BlockSpec((1, H), lambda i: (0, 0)),
            pl.BlockSpec((1, P), lambda i: (0, 0)),
            pl.BlockSpec((1, P), lambda i: (0, 0)),
        ],
        out_specs=pl.BlockSpec((_T, H), lambda i: (i, 0)),
        scratch_shapes=[
            pltpu.VMEM((8, P), jnp.float32),
            pltpu.VMEM((2 * H, 2 * P), jnp.float32),
            pltpu.VMEM((2 * H, 2 * P), jnp.float32),
            pltpu.VMEM((2 * P, 2 * H), jnp.float32),
            pltpu.VMEM((2 * P, 2 * H), jnp.float32),
        ],
        compiler_params=pltpu.CompilerParams(
            dimension_semantics=("arbitrary",),
        ),
        name="linoss_scan",
    )(
        input_sequence,
        B_real.T, B_img.T,
        C_real.T, C_img.T,
        D.reshape(1, H),
        A_diag_raw.reshape(1, P),
        steps_raw.reshape(1, P),
    )
